# baseline (speedup 1.0000x reference)
import functools
import math

import jax
import jax.numpy as jnp
import numpy as np
from jax import lax
from jax.experimental import pallas as pl
from jax.experimental.pallas import tpu as pltpu
from jax.experimental.pallas import tpu_sc as plsc

K = 1000
B = 16384

NC = 2
NS = 16
NW = NC * NS
RPW = B // NW
CHK = 128
NCHK = RPW // CHK

_SCALE = np.float32(2.0 * math.pi / K)


def _r_kernel(o_ref):
    i = lax.broadcasted_iota(jnp.int32, (K, K), 0)
    k = lax.broadcasted_iota(jnp.int32, (K, K), 1)
    d = jnp.abs(i - k)
    o_ref[...] = jnp.minimum(d, K - d).astype(jnp.float32) * (-_SCALE)


def _build_r():
    return pl.pallas_call(
        _r_kernel,
        out_shape=jax.ShapeDtypeStruct((K, K), jnp.float32),
    )()


@functools.partial(
    pl.kernel,
    mesh=plsc.VectorSubcoreMesh(core_axis_name="c", subcore_axis_name="s"),
    out_type=jax.ShapeDtypeStruct((B, K), jnp.float32),
    compiler_params=pltpu.CompilerParams(use_tc_tiling_on_sc=False),
    scratch_types=[
        pltpu.VMEM((CHK,), jnp.int32),
        pltpu.VMEM((CHK,), jnp.int32),
        pltpu.VMEM((CHK,), jnp.int32),
        pltpu.VMEM((CHK, K), jnp.float32),
        pltpu.SemaphoreType.DMA,
    ],
)
def _sc_lookup(x1_hbm, x2_hbm, r_hbm, out_hbm, xa_v, xb_v, idx_v, rows_v, sem):
    wid = lax.axis_index("s") * NC + lax.axis_index("c")

    def chunk(c, _):
        base = wid * RPW + c * CHK
        pltpu.sync_copy(x1_hbm.at[pl.ds(base, CHK)], xa_v)
        pltpu.sync_copy(x2_hbm.at[pl.ds(base, CHK)], xb_v)
        for j in range(CHK // 16):
            s = xa_v[pl.ds(j * 16, 16)] + xb_v[pl.ds(j * 16, 16)]
            idx_v[pl.ds(j * 16, 16)] = jnp.where(s >= K, s - K, s)
        pltpu.async_copy(r_hbm.at[idx_v], rows_v, sem).wait()
        pltpu.sync_copy(rows_v, out_hbm.at[pl.ds(base, CHK)])
        return 0

    lax.fori_loop(0, NCHK, chunk, 0)


def kernel(x1, x2, input_phases, output_phases):
    del input_phases, output_phases
    r = _build_r()
    return _sc_lookup(x1.astype(jnp.int32), x2.astype(jnp.int32), r)


# speedup vs baseline: 2.0402x; 2.0402x over previous
import math

import jax
import jax.numpy as jnp
import numpy as np
from jax.experimental import pallas as pl
from jax.experimental.pallas import tpu as pltpu

K = 1000
KF = 896
B = 16384
CH = 1024
NBUF = 4
NSTEPS = B // CH

_TWO_PI = np.float32(2.0 * math.pi)
_SCALE = np.float32(2.0 * math.pi / K)


def _dist_kernel(x1_ref, x2_ref, op_ref, o_ref, scratch, semf, semp):
    opv = op_ref[...]

    def copies(i, slot):
        rows = pl.ds(i * CH, CH)
        return (
            pltpu.make_async_copy(
                scratch.at[slot, :, pl.ds(0, KF)],
                o_ref.at[rows, pl.ds(0, KF)], semf.at[slot]),
            pltpu.make_async_copy(
                scratch.at[slot, :, pl.ds(KF, K - KF)],
                o_ref.at[rows, pl.ds(KF, K - KF)], semp.at[slot]),
        )

    def body(i, _):
        slot = jax.lax.rem(i, NBUF)

        @pl.when(i >= NBUF)
        def _wait_prev():
            cf, cp = copies(i, slot)
            cf.wait()
            cp.wait()

        p1 = x1_ref[pl.ds(i * CH, CH), :].astype(jnp.float32) * _SCALE
        p2 = x2_ref[pl.ds(i * CH, CH), :].astype(jnp.float32) * _SCALE
        t = p1 + p2
        phi = jnp.where(t >= _TWO_PI, t - _TWO_PI, t)
        d = jnp.abs(phi - opv)
        scratch[slot] = -jnp.minimum(d, _TWO_PI - d)
        cf, cp = copies(i, slot)
        cf.start()
        cp.start()
        return 0

    jax.lax.fori_loop(0, NSTEPS, body, 0)

    def drain(i, _):
        j = NSTEPS - NBUF + i
        cf, cp = copies(j, jax.lax.rem(j, NBUF))
        cf.wait()
        cp.wait()
        return 0

    jax.lax.fori_loop(0, NBUF, drain, 0)


def kernel(x1, x2, input_phases, output_phases):
    del input_phases
    x1c = x1.astype(jnp.int32).reshape(B, 1)
    x2c = x2.astype(jnp.int32).reshape(B, 1)
    opr = output_phases.reshape(1, K)
    return pl.pallas_call(
        _dist_kernel,
        in_specs=[
            pl.BlockSpec(memory_space=pltpu.MemorySpace.VMEM),
            pl.BlockSpec(memory_space=pltpu.MemorySpace.VMEM),
            pl.BlockSpec(memory_space=pltpu.MemorySpace.VMEM),
        ],
        out_specs=pl.BlockSpec(memory_space=pl.ANY),
        out_shape=jax.ShapeDtypeStruct((B, K), jnp.float32),
        scratch_shapes=[
            pltpu.VMEM((NBUF, CH, K), jnp.float32),
            pltpu.SemaphoreType.DMA((NBUF,)),
            pltpu.SemaphoreType.DMA((NBUF,)),
        ],
    )(x1c, x2c, opr)
